# baseline (speedup 1.0000x reference)
import functools
import math

import jax
import jax.numpy as jnp
from jax import lax
from jax.experimental import pallas as pl
from jax.experimental.pallas import tpu as pltpu
from jax.experimental.pallas import tpu_sc as plsc

_B, _N, _D = 8, 512, 128
_IOU_THRESHOLD = 0.8
_INV_TEMP = 10.0
_LOG2E = math.log2(math.e)
_SIDE_SCALE = math.sqrt(_INV_TEMP * _LOG2E)

_NC, _NS, _LANES = 2, 16, 16
_NW = _NC * _NS
_SEG = 128
_BT = _B * _N
_BPW = _BT // _NW


def _phase1_kernel(feat_a_ref, feat_b_ref, iou_ref,
                   an_ref, bn_ref, m_ref, ridx_ref, lane_ref, rowmax_ref):
    b = pl.program_id(0)

    @pl.when(b == 0)
    def _():
        fa = feat_a_ref[...].reshape(_B * _N, _D)
        fb = feat_b_ref[...].reshape(_B * _N, _D)
        na = jnp.sqrt(jnp.sum(fa * fa, axis=-1, keepdims=True)) + 1e-8
        nb = jnp.sqrt(jnp.sum(fb * fb, axis=-1, keepdims=True)) + 1e-8
        an_ref[...] = (fa * (_SIDE_SCALE / na)).astype(jnp.float8_e4m3fn)
        bn_ref[...] = (fb * (_SIDE_SCALE / nb)).astype(jnp.float8_e4m3fn)

    iou_b = iou_ref[0]
    rowmax = jnp.max(iou_b, axis=-1, keepdims=True)
    col = lax.broadcasted_iota(jnp.int32, (_N, _N), 1)
    eq = iou_b == rowmax
    idx = jnp.min(jnp.where(eq, col, _N), axis=-1, keepdims=True)
    onehot = (col == idx).astype(jnp.bfloat16)

    an_b = an_ref[pl.ds(b * _N, _N), :]
    match = jnp.dot(onehot, an_b.astype(jnp.bfloat16),
                    preferred_element_type=jnp.float32)
    m_ref[...] = match.astype(jnp.float8_e4m3fn)

    riota = lax.broadcasted_iota(jnp.int32, (_N, 1), 0)
    ridx_ref[...] = (b * (_N * _N // _SEG) + riota * (_N // _SEG)
                     + lax.shift_right_logical(idx, 7))[None]
    lane_ref[...] = jnp.bitwise_and(idx, _SEG - 1)[None]
    rowmax_ref[...] = rowmax[None]


def _loop_kernel(m_ref, an_ref, bn_ref, negsum_ref):
    b = pl.program_id(0)
    m8 = m_ref[...]
    acc = jnp.zeros((_N, _D), jnp.float32)
    for j in range(_B - 1):
        jj = j + (j >= b).astype(jnp.int32)
        a_j = an_ref[pl.ds(jj * _N, _N), :]
        b_j = bn_ref[pl.ds(jj * _N, _N), :]
        ga = lax.dot_general(m8, a_j, (((1,), (1,)), ((), ())),
                             preferred_element_type=jnp.float32)
        gb = lax.dot_general(m8, b_j, (((1,), (1,)), ((), ())),
                             preferred_element_type=jnp.float32)
        ea = jnp.exp2(ga.astype(jnp.bfloat16))
        eb = jnp.exp2(gb.astype(jnp.bfloat16))
        sa = ((ea[:, 0:128] + ea[:, 128:256])
              + (ea[:, 256:384] + ea[:, 384:512]))
        sb = ((eb[:, 0:128] + eb[:, 128:256])
              + (eb[:, 256:384] + eb[:, 384:512]))
        acc = acc + (sa.astype(jnp.float32) + sb.astype(jnp.float32))
    negsum_ref[...] = jnp.sum(acc, axis=-1, keepdims=True)[None]


def _combine_kernel(negsum_ref, segs_ref, lane_ref, rowmax_ref,
                    loss_ref, cnt_ref):
    segs = segs_ref[...]
    lane = lane_ref[...].reshape(_B * _N, 1)
    laneiota = lax.broadcasted_iota(jnp.int32, (_BT, _SEG), 1)
    pos = jnp.sum(jnp.where(laneiota == lane, segs, 0.0),
                  axis=-1, keepdims=True)
    negsum = negsum_ref[...].reshape(_BT, 1)
    total = negsum + jnp.exp2(pos * (_INV_TEMP * _LOG2E))
    row_loss = jnp.log(total) - pos * _INV_TEMP
    rm = (rowmax_ref[...].reshape(_BT, 1) >= _IOU_THRESHOLD
          ).astype(jnp.float32)
    rl = (row_loss * rm).reshape(_B, _N)
    cnt = jnp.sum(rm.reshape(_B, _N), axis=-1, keepdims=True)
    loss_ref[...] = (jnp.sum(rl, axis=-1, keepdims=True) / cnt)[:, :, None]
    cnt_ref[...] = cnt.astype(jnp.int32)[:, :, None]


def _sc_gather(sim_rows, ridx):
    mesh = plsc.VectorSubcoreMesh(core_axis_name="c", subcore_axis_name="s")

    @functools.partial(
        pl.kernel, mesh=mesh,
        out_type=jax.ShapeDtypeStruct((_BT, _SEG), jnp.float32),
        scratch_types=[
            pltpu.VMEM((_BPW,), jnp.int32),
            pltpu.VMEM((_BPW, _SEG), jnp.float32),
            pltpu.SemaphoreType.DMA,
        ],
    )
    def k(table_hbm, idx_hbm, out_hbm, idx_v, rows_v, sem):
        wid = lax.axis_index("s") * _NC + lax.axis_index("c")
        base = wid * _BPW
        pltpu.sync_copy(idx_hbm.at[pl.ds(base, _BPW)], idx_v)
        pltpu.async_copy(table_hbm.at[idx_v], rows_v, sem).wait()
        pltpu.sync_copy(rows_v, out_hbm.at[pl.ds(base, _BPW)])

    return k(sim_rows, ridx)


def kernel(feat_a, feat_b, sim, iou):
    an, bn, mtab, ridx, lane, rowmax = pl.pallas_call(
        _phase1_kernel,
        grid=(_B,),
        in_specs=[
            pl.BlockSpec((_B, _N, _D), lambda b: (0, 0, 0)),
            pl.BlockSpec((_B, _N, _D), lambda b: (0, 0, 0)),
            pl.BlockSpec((1, _N, _N), lambda b: (b, 0, 0)),
        ],
        out_specs=[
            pl.BlockSpec((_B * _N, _D), lambda b: (0, 0)),
            pl.BlockSpec((_B * _N, _D), lambda b: (0, 0)),
            pl.BlockSpec((_N, _D), lambda b: (b, 0)),
            pl.BlockSpec((1, _N, 1), lambda b: (b, 0, 0)),
            pl.BlockSpec((1, _N, 1), lambda b: (b, 0, 0)),
            pl.BlockSpec((1, _N, 1), lambda b: (b, 0, 0)),
        ],
        out_shape=[
            jax.ShapeDtypeStruct((_B * _N, _D), jnp.float8_e4m3fn),
            jax.ShapeDtypeStruct((_B * _N, _D), jnp.float8_e4m3fn),
            jax.ShapeDtypeStruct((_B * _N, _D), jnp.float8_e4m3fn),
            jax.ShapeDtypeStruct((_B, _N, 1), jnp.int32),
            jax.ShapeDtypeStruct((_B, _N, 1), jnp.int32),
            jax.ShapeDtypeStruct((_B, _N, 1), jnp.float32),
        ],
    )(feat_a, feat_b, iou)

    segs = _sc_gather(sim.reshape(_B * _N * _N // _SEG, _SEG),
                      ridx.reshape(_BT))

    negsum = pl.pallas_call(
        _loop_kernel,
        grid=(_B,),
        in_specs=[
            pl.BlockSpec((_N, _D), lambda b: (b, 0)),
            pl.BlockSpec((_B * _N, _D), lambda b: (0, 0)),
            pl.BlockSpec((_B * _N, _D), lambda b: (0, 0)),
        ],
        out_specs=pl.BlockSpec((1, _N, 1), lambda b: (b, 0, 0)),
        out_shape=jax.ShapeDtypeStruct((_B, _N, 1), jnp.float32),
    )(mtab, an, bn)

    loss, cnt = pl.pallas_call(
        _combine_kernel,
        in_specs=[
            pl.BlockSpec((_B, _N, 1), lambda: (0, 0, 0)),
            pl.BlockSpec((_BT, _SEG), lambda: (0, 0)),
            pl.BlockSpec((_B, _N, 1), lambda: (0, 0, 0)),
            pl.BlockSpec((_B, _N, 1), lambda: (0, 0, 0)),
        ],
        out_specs=[
            pl.BlockSpec((_B, 1, 1), lambda: (0, 0, 0)),
            pl.BlockSpec((_B, 1, 1), lambda: (0, 0, 0)),
        ],
        out_shape=[
            jax.ShapeDtypeStruct((_B, 1, 1), jnp.float32),
            jax.ShapeDtypeStruct((_B, 1, 1), jnp.int32),
        ],
    )(negsum, segs, lane, rowmax)
    return (loss[:, 0, 0], cnt[:, 0, 0])


# speedup vs baseline: 2.1234x; 2.1234x over previous
import math

import jax
import jax.numpy as jnp
from jax import lax
from jax.experimental import pallas as pl
from jax.experimental.pallas import tpu as pltpu

_B, _N, _D = 8, 512, 128
_IOU_THRESHOLD = 0.8
_INV_TEMP = 10.0
_LOG2E = math.log2(math.e)
_SIDE_SCALE = math.sqrt(_INV_TEMP * _LOG2E)


def _one_batch(b, iou_b, sim_b, an_ref, bn_ref):
    rowmax = jnp.max(iou_b, axis=-1, keepdims=True)
    col = lax.broadcasted_iota(jnp.int32, (_N, _N), 1)
    eq = iou_b == rowmax
    idx = jnp.min(jnp.where(eq, col, _N), axis=-1, keepdims=True)
    onehot = (col == idx).astype(jnp.float32)
    pos = jnp.sum(onehot * sim_b, axis=-1)

    an_b = an_ref[pl.ds(b * _N, _N), :]
    match = jnp.dot(onehot.astype(jnp.bfloat16), an_b.astype(jnp.bfloat16),
                    preferred_element_type=jnp.float32)
    m8 = match.astype(jnp.float8_e4m3fn)

    acc = jnp.zeros((_N, _D), jnp.float32)
    for j in range(_B - 1):
        jj = j + (j >= b).astype(jnp.int32)
        a_j = an_ref[pl.ds(jj * _N, _N), :]
        b_j = bn_ref[pl.ds(jj * _N, _N), :]
        ga = lax.dot_general(m8, a_j, (((1,), (1,)), ((), ())),
                             preferred_element_type=jnp.float32)
        gb = lax.dot_general(m8, b_j, (((1,), (1,)), ((), ())),
                             preferred_element_type=jnp.float32)
        ea = jnp.exp2(ga.astype(jnp.bfloat16))
        eb = jnp.exp2(gb.astype(jnp.bfloat16))
        sa = ((ea[:, 0:128] + ea[:, 128:256])
              + (ea[:, 256:384] + ea[:, 384:512]))
        sb = ((eb[:, 0:128] + eb[:, 128:256])
              + (eb[:, 256:384] + eb[:, 384:512]))
        acc = acc + (sa.astype(jnp.float32) + sb.astype(jnp.float32))
    total = jnp.sum(acc, axis=-1) + jnp.exp2(pos * (_INV_TEMP * _LOG2E))

    row_loss = jnp.log(total) - pos * _INV_TEMP
    rm = (rowmax[:, 0] >= _IOU_THRESHOLD).astype(jnp.float32)
    cnt = jnp.sum(rm)
    return jnp.sum(row_loss * rm) / cnt, cnt.astype(jnp.int32)


def _loss_kernel(feat_a_ref, feat_b_ref, sim_ref, iou_ref,
                 loss_ref, cnt_ref, an_ref, bn_ref):
    g = pl.program_id(0)

    @pl.when(g == 0)
    def _():
        fa = feat_a_ref[...].reshape(_B * _N, _D)
        fb = feat_b_ref[...].reshape(_B * _N, _D)
        na = jnp.sqrt(jnp.sum(fa * fa, axis=-1, keepdims=True)) + 1e-8
        nb = jnp.sqrt(jnp.sum(fb * fb, axis=-1, keepdims=True)) + 1e-8
        an_ref[...] = (fa * (_SIDE_SCALE / na)).astype(jnp.float8_e4m3fn)
        bn_ref[...] = (fb * (_SIDE_SCALE / nb)).astype(jnp.float8_e4m3fn)

    l0, c0 = _one_batch(g, iou_ref[0], sim_ref[0], an_ref, bn_ref)
    loss_ref[...] = l0[None, None, None]
    cnt_ref[...] = c0[None, None, None]


def kernel(feat_a, feat_b, sim, iou):
    loss, cnt = pl.pallas_call(
        _loss_kernel,
        grid=(_B,),
        in_specs=[
            pl.BlockSpec((_B, _N, _D), lambda g: (0, 0, 0)),
            pl.BlockSpec((_B, _N, _D), lambda g: (0, 0, 0)),
            pl.BlockSpec((1, _N, _N), lambda g: (g, 0, 0)),
            pl.BlockSpec((1, _N, _N), lambda g: (g, 0, 0)),
        ],
        out_specs=[
            pl.BlockSpec((1, 1, 1), lambda g: (g, 0, 0)),
            pl.BlockSpec((1, 1, 1), lambda g: (g, 0, 0)),
        ],
        out_shape=[
            jax.ShapeDtypeStruct((_B, 1, 1), jnp.float32),
            jax.ShapeDtypeStruct((_B, 1, 1), jnp.int32),
        ],
        scratch_shapes=[
            pltpu.VMEM((_B * _N, _D), jnp.float8_e4m3fn),
            pltpu.VMEM((_B * _N, _D), jnp.float8_e4m3fn),
        ],
    )(feat_a, feat_b, sim, iou)
    return (loss[:, 0, 0], cnt[:, 0, 0])
